# baseline (speedup 1.0000x reference)
import functools

import jax
import jax.numpy as jnp
from jax import lax
from jax.experimental import pallas as pl
from jax.experimental.pallas import tpu as pltpu
from jax.experimental.pallas import tpu_sc as plsc

TOTAL = 64 * 160000
NC, NS = 2, 16

SP_CH = 409600
SP_RB = 3
SP_K = 5
SP_PER_SC = SP_CH * SP_K
SP_TOTAL = SP_PER_SC * NC

NSW = (NS - 1) * NC
ST_TOTAL = TOTAL - SP_TOTAL
ST_PER_W = ST_TOTAL // NSW
ST_CH = 12800
ST_NCHUNK = ST_PER_W // ST_CH
ST_NBUF = 4

_mesh = plsc.VectorSubcoreMesh(core_axis_name="c", subcore_axis_name="s")


@functools.partial(
    pl.kernel,
    mesh=_mesh,
    out_type=jax.ShapeDtypeStruct((TOTAL,), jnp.float32),
    scratch_types=[
        pltpu.VMEM((ST_NBUF * ST_CH,), jnp.float32),
        pltpu.MemorySpace.VMEM_SHARED((SP_RB * SP_CH,), jnp.float32),
        pltpu.SemaphoreType.DMA,
        pltpu.SemaphoreType.DMA,
        pltpu.SemaphoreType.DMA,
        pltpu.SemaphoreType.DMA,
        pltpu.SemaphoreType.DMA,
        pltpu.SemaphoreType.DMA,
        pltpu.SemaphoreType.DMA,
        pltpu.SemaphoreType.DMA,
        pltpu.SemaphoreType.DMA,
        pltpu.SemaphoreType.DMA,
        pltpu.SemaphoreType.DMA,
        pltpu.SemaphoreType.DMA,
        pltpu.SemaphoreType.DMA,
        pltpu.SemaphoreType.DMA,
        pltpu.SemaphoreType.DMA,
        pltpu.SemaphoreType.DMA,
    ],
)
def _sc_copy(x_hbm, o_hbm, buf, spbuf, *sems):
    c = lax.axis_index("c")
    s = lax.axis_index("s")
    st_in_sems = sems[:ST_NBUF]
    st_out_sems = sems[ST_NBUF:2 * ST_NBUF]
    sp_in_sems = sems[2 * ST_NBUF:2 * ST_NBUF + SP_RB]
    sp_out_sems = sems[2 * ST_NBUF + SP_RB:2 * ST_NBUF + 2 * SP_RB]

    @pl.when(s == 0)
    def _spmem_path():
        spbase = c * SP_PER_SC

        def sp_in(k):
            return pltpu.make_async_copy(
                x_hbm.at[pl.ds(spbase + k * SP_CH, SP_CH)],
                spbuf.at[pl.ds((k % SP_RB) * SP_CH, SP_CH)], sp_in_sems[k % SP_RB])

        def sp_out(k):
            return pltpu.make_async_copy(
                spbuf.at[pl.ds((k % SP_RB) * SP_CH, SP_CH)],
                o_hbm.at[pl.ds(spbase + k * SP_CH, SP_CH)],
                sp_out_sems[k % SP_RB])

        for k in range(min(SP_RB, SP_K)):
            sp_in(k).start()
        for k in range(SP_K):
            sp_in(k).wait()
            sp_out(k).start()
            nxt = k + SP_RB
            if nxt < SP_K:
                sp_out(k).wait()
                sp_in(nxt).start()
        for k in range(max(0, SP_K - SP_RB), SP_K):
            sp_out(k).wait()

    @pl.when(s != 0)
    def _stream_path():
        swid = (s - 1) * NC + c
        base = SP_TOTAL + swid * ST_PER_W

        def st_in(k):
            return pltpu.make_async_copy(
                x_hbm.at[pl.ds(base + k * ST_CH, ST_CH)],
                buf.at[pl.ds((k % ST_NBUF) * ST_CH, ST_CH)], st_in_sems[k % ST_NBUF])

        def st_out(k):
            return pltpu.make_async_copy(
                buf.at[pl.ds((k % ST_NBUF) * ST_CH, ST_CH)],
                o_hbm.at[pl.ds(base + k * ST_CH, ST_CH)],
                st_out_sems[k % ST_NBUF])

        for k in range(ST_NBUF):
            st_in(k).start()
        for k in range(ST_NCHUNK):
            st_in(k).wait()
            st_out(k).start()
            nxt = k + ST_NBUF
            if nxt < ST_NCHUNK:
                st_out(k).wait()
                st_in(nxt).start()
        for k in range(ST_NCHUNK - ST_NBUF, ST_NCHUNK):
            st_out(k).wait()


def kernel(samples, sample_rate):
    x = samples.reshape(TOTAL)
    out = _sc_copy(x)
    return out.reshape(samples.shape)
